# baseline (speedup 1.0000x reference)
import functools

import jax
import jax.numpy as jnp
from jax import lax
from jax.experimental import pallas as pl
from jax.experimental.pallas import tpu as pltpu
from jax.experimental.pallas import tpu_sc as plsc

_D = 64
_H = 50
_BATCH = 16384
_NW = 32
_BB = 128
_NCHUNK = _H * (_BATCH // _BB)
_CPW = _NCHUNK // _NW
_NBUF = 2


def _gather_sc(idx_grp, table):
    mesh = plsc.VectorSubcoreMesh(core_axis_name="c", subcore_axis_name="s")

    @functools.partial(
        pl.kernel,
        mesh=mesh,
        out_type=jax.ShapeDtypeStruct((_H, 8, _BATCH // _BB, 8, _BB), jnp.float32),
        compiler_params=pltpu.CompilerParams(
            use_tc_tiling_on_sc=False, needs_layout_passes=False
        ),
        scratch_types=[
            pltpu.VMEM((_CPW, _BB), jnp.int32),
            pltpu.VMEM((_NBUF, _BB, _D), jnp.float32),
            pltpu.VMEM((_D, _BB), jnp.float32),
            pltpu.SemaphoreType.DMA,
            pltpu.SemaphoreType.DMA,
        ],
    )
    def k(idx_hbm, table_hbm, out_hbm, idx_v, rows_v, t_v, gsem0, gsem1):
        gsems = (gsem0, gsem1)
        wid = lax.axis_index("s") * 2 + lax.axis_index("c")
        base_c = wid * _CPW
        pltpu.sync_copy(idx_hbm.at[wid], idx_v)

        def start_gather(i, nb):
            pltpu.async_copy(
                table_hbm.at[idx_v.at[i]],
                rows_v.at[nb],
                gsems[nb],
            )

        def wait_gather(i, nb):
            pltpu.make_async_copy(
                table_hbm.at[idx_v.at[i]],
                rows_v.at[nb],
                gsems[nb],
            ).wait()

        bvecs = [lax.iota(jnp.int32, 16) + j * 16 for j in range(8)]

        for nb in range(_NBUF):
            start_gather(nb, nb)

        def body(i, carry):
            for nb in range(_NBUF):
                ic = i * _NBUF + nb
                c = base_c + ic
                h = c // (_BATCH // _BB)
                bblk = c % (_BATCH // _BB)
                wait_gather(ic, nb)

                def trans_d(d, _):
                    dvec = jnp.full((16,), d, dtype=jnp.int32)
                    for j in range(8):
                        vals = plsc.load_gather(rows_v.at[nb], [bvecs[j], dvec])
                        t_v[d, pl.ds(j * 16, 16)] = vals
                    return _

                lax.fori_loop(0, _D, trans_d, 0)

                @pl.when(ic + _NBUF < _CPW)
                def _():
                    start_gather(ic + _NBUF, nb)

                for dblk in range(8):
                    pltpu.sync_copy(
                        t_v.at[pl.ds(dblk * 8, 8)],
                        out_hbm.at[h, dblk, bblk],
                    )

            return carry

        lax.fori_loop(0, _CPW // _NBUF, body, 0)

    return k(idx_grp, table)


def kernel(input, table):
    idxt = jnp.transpose(input.astype(jnp.int32))
    idx_grp = idxt.reshape(_NW, _CPW, _BB)
    out5 = _gather_sc(idx_grp, table)
    out = jnp.transpose(out5, (2, 4, 0, 1, 3)).reshape(_BATCH, _H, _D)
    return out


# speedup vs baseline: 1.0797x; 1.0797x over previous
import functools

import jax
import jax.numpy as jnp
from jax import lax
from jax.experimental import pallas as pl
from jax.experimental.pallas import tpu as pltpu
from jax.experimental.pallas import tpu_sc as plsc

_D = 64
_H = 50
_BATCH = 16384
_NW = 32
_BB = 128
_NCHUNK = _H * (_BATCH // _BB)
_CPW = _NCHUNK // _NW
_NBUF = 2


def _gather_sc(idx_grp, table):
    mesh = plsc.VectorSubcoreMesh(core_axis_name="c", subcore_axis_name="s")

    @functools.partial(
        pl.kernel,
        mesh=mesh,
        out_type=jax.ShapeDtypeStruct((_H, 8, _BATCH // _BB, 8, _BB), jnp.float32),
        compiler_params=pltpu.CompilerParams(
            use_tc_tiling_on_sc=False, needs_layout_passes=False
        ),
        scratch_types=[
            pltpu.VMEM((_CPW, _BB), jnp.int32),
            pltpu.VMEM((_NBUF, _BB, _D), jnp.float32),
            pltpu.VMEM((_NBUF, _D, _BB), jnp.float32),
            pltpu.SemaphoreType.DMA,
            pltpu.SemaphoreType.DMA,
            pltpu.SemaphoreType.DMA,
            pltpu.SemaphoreType.DMA,
        ],
    )
    def k(idx_hbm, table_hbm, out_hbm, idx_v, rows_v, t_v, gsem0, gsem1, osem0, osem1):
        gsems = (gsem0, gsem1)
        osems = (osem0, osem1)
        nbb = _BATCH // _BB
        wid = lax.axis_index("s") * 2 + lax.axis_index("c")
        base_c = wid * _CPW
        pltpu.sync_copy(idx_hbm.at[wid], idx_v)

        def start_gather(i, nb):
            pltpu.async_copy(
                table_hbm.at[idx_v.at[i]],
                rows_v.at[nb],
                gsems[nb],
            )

        def wait_gather(i, nb):
            pltpu.make_async_copy(
                table_hbm.at[idx_v.at[i]],
                rows_v.at[nb],
                gsems[nb],
            ).wait()

        bvecs = [lax.iota(jnp.int32, 16) + j * 16 for j in range(8)]

        for nb in range(_NBUF):
            start_gather(nb, nb)

        def drain_out(c2, nb):
            h2 = c2 // nbb
            bb2 = c2 % nbb
            for dblk in range(8):
                pltpu.make_async_copy(
                    t_v.at[nb].at[pl.ds(dblk * 8, 8)],
                    out_hbm.at[h2, dblk, bb2],
                    osems[nb],
                ).wait()

        def body(i, carry):
            for nb in range(_NBUF):
                ic = i * _NBUF + nb
                c = base_c + ic
                h = c // nbb
                bblk = c % nbb
                wait_gather(ic, nb)

                @pl.when(ic >= _NBUF)
                def _():
                    drain_out(c - _NBUF, nb)

                def trans_d(dd, _):
                    for u in range(4):
                        d = dd * 4 + u
                        dvec = jnp.full((16,), d, dtype=jnp.int32)
                        for j in range(8):
                            vals = plsc.load_gather(
                                rows_v.at[nb], [bvecs[j], dvec]
                            )
                            t_v[nb, d, pl.ds(j * 16, 16)] = vals
                    return _

                lax.fori_loop(0, _D // 4, trans_d, 0)

                @pl.when(ic + _NBUF < _CPW)
                def _():
                    start_gather(ic + _NBUF, nb)

                for dblk in range(8):
                    pltpu.async_copy(
                        t_v.at[nb].at[pl.ds(dblk * 8, 8)],
                        out_hbm.at[h, dblk, bblk],
                        osems[nb],
                    )

            return carry

        lax.fori_loop(0, _CPW // _NBUF, body, 0)
        for nb in range(_NBUF):
            drain_out(base_c + _CPW - _NBUF + nb, nb)

    return k(idx_grp, table)


def kernel(input, table):
    idxt = jnp.transpose(input.astype(jnp.int32))
    idx_grp = idxt.reshape(_NW, _CPW, _BB)
    out5 = _gather_sc(idx_grp, table)
    out = jnp.transpose(out5, (2, 4, 0, 1, 3)).reshape(_BATCH, _H, _D)
    return out


# speedup vs baseline: 1.5730x; 1.4570x over previous
import functools

import jax
import jax.numpy as jnp
from jax import lax
from jax.experimental import pallas as pl
from jax.experimental.pallas import tpu as pltpu
from jax.experimental.pallas import tpu_sc as plsc

_D = 64
_NW = 32
_B = 16384 * 50
_BPW = _B // _NW
_SUB = 128
_SPC = 4
_CH = _SUB * _SPC
_NCH = _BPW // _CH
_NBUF = 2


def _gather_sc(idx_grp, table):
    mesh = plsc.VectorSubcoreMesh(core_axis_name="c", subcore_axis_name="s")

    @functools.partial(
        pl.kernel,
        mesh=mesh,
        out_type=jax.ShapeDtypeStruct((_B, _D), jnp.float32),
        compiler_params=pltpu.CompilerParams(use_tc_tiling_on_sc=False),
        scratch_types=[
            pltpu.VMEM((_NCH * _SPC, _SUB), jnp.int32),
            pltpu.VMEM((_NBUF, _CH, _D), jnp.float32),
            pltpu.SemaphoreType.DMA,
            pltpu.SemaphoreType.DMA,
        ],
    )
    def k(idx_hbm, table_hbm, out_hbm, idx_v, rows_v, gsem0, gsem1):
        gsems = (gsem0, gsem1)
        wid = lax.axis_index("s") * 2 + lax.axis_index("c")
        base = wid * _BPW
        pltpu.sync_copy(idx_hbm.at[wid], idx_v)

        def start_gather(g, b):
            for j in range(_SPC):
                pltpu.async_copy(
                    table_hbm.at[idx_v.at[g * _SPC + j]],
                    rows_v.at[b].at[pl.ds(j * _SUB, _SUB)],
                    gsems[b],
                )

        def wait_gather(g, b):
            for j in range(_SPC):
                pltpu.make_async_copy(
                    table_hbm.at[idx_v.at[g * _SPC + j]],
                    rows_v.at[b].at[pl.ds(j * _SUB, _SUB)],
                    gsems[b],
                ).wait()

        for b in range(_NBUF):
            start_gather(b, b)

        def body(t, carry):
            for b in range(_NBUF):
                g = t * _NBUF + b
                wait_gather(g, b)
                pltpu.sync_copy(
                    rows_v.at[b], out_hbm.at[pl.ds(base + g * _CH, _CH)]
                )

                @pl.when(g + _NBUF < _NCH)
                def _():
                    start_gather(g + _NBUF, b)

            return carry

        lax.fori_loop(0, _NCH // _NBUF, body, 0)

    return k(idx_grp, table)


def kernel(input, table):
    idx = input.reshape(-1).astype(jnp.int32).reshape(_NW, _NCH * _SPC, _SUB)
    out = _gather_sc(idx, table)
    return out.reshape(input.shape[0], input.shape[1], _D)
